# baseline (speedup 1.0000x reference)
import functools

import jax
import jax.numpy as jnp
from jax import lax
from jax.experimental import pallas as pl
from jax.experimental.pallas import tpu as pltpu
from jax.experimental.pallas import tpu_sc as plsc

_E = 8
_K = 2
_M = 256
_BH = 512
_TB = 256


def _gate_body(x_ref, wg_ref, ts_ref, ti_ref):
    s = lax.dot_general(x_ref[...], wg_ref[...], (((1,), (1,)), ((), ())),
                        preferred_element_type=jnp.float32)
    iota = lax.broadcasted_iota(jnp.int32, s.shape, 1)
    v1 = jnp.max(s, axis=1, keepdims=True)
    i1 = jnp.min(jnp.where(s == v1, iota, _E), axis=1, keepdims=True)
    s2 = jnp.where(iota == i1, -jnp.inf, s)
    v2 = jnp.max(s2, axis=1, keepdims=True)
    i2 = jnp.min(jnp.where(s2 == v2, iota, _E), axis=1, keepdims=True)
    ts_ref[...] = jnp.concatenate([v1, v2], axis=1)
    ti_ref[...] = jnp.concatenate([i1, i2], axis=1)


def _gate_topk(x2d, Wg):
    T, D = x2d.shape
    return pl.pallas_call(
        _gate_body,
        grid=(T // _TB,),
        in_specs=[
            pl.BlockSpec((_TB, D), lambda i: (i, 0)),
            pl.BlockSpec((_E, D), lambda i: (0, 0)),
        ],
        out_specs=[
            pl.BlockSpec((_TB, _K), lambda i: (i, 0)),
            pl.BlockSpec((_TB, _K), lambda i: (i, 0)),
        ],
        out_shape=[
            jax.ShapeDtypeStruct((T, _K), jnp.float32),
            jax.ShapeDtypeStruct((T, _K), jnp.int32),
        ],
    )(x2d, Wg)


def _routing_metadata(ts, ti, P):
    T = ts.shape[0]
    TK = T * _K
    a = ti.reshape(-1)
    oh = (a[:, None] == jnp.arange(_E, dtype=jnp.int32)).astype(jnp.int32)
    csum = jnp.cumsum(oh, axis=0)
    cnt = csum[-1]
    pc = ((cnt + _M - 1) // _M) * _M
    start = jnp.concatenate(
        [jnp.zeros(1, jnp.int32), jnp.cumsum(pc)[:-1].astype(jnp.int32)])
    rank = jnp.take_along_axis(csum, a[:, None], axis=1)[:, 0] - 1
    pos = start[a] + rank
    tok = jnp.arange(TK, dtype=jnp.int32) // _K
    src = jnp.zeros(P, jnp.int32).at[pos].set(tok)
    wvec = jnp.zeros(P, jnp.float32).at[pos].set(ts.reshape(-1))
    tile_base = jnp.arange(P // _M, dtype=jnp.int32) * _M
    eid = jnp.clip(jnp.searchsorted(start, tile_base, side="right") - 1,
                   0, _E - 1).astype(jnp.int32)
    pos2 = pos.reshape(T, _K)
    return src, wvec, eid, pos2[:, 0], pos2[:, 1]


def _ffn_body(eid_ref, xs_ref, w1_ref, w2_ref, w3_ref, wv_ref, ys_ref,
              acc_ref, *, nh):
    del eid_ref
    j = pl.program_id(1)
    x = xs_ref[...]
    g = lax.dot_general(x, w1_ref[0], (((1,), (1,)), ((), ())),
                        preferred_element_type=jnp.float32)
    u = lax.dot_general(x, w2_ref[0], (((1,), (1,)), ((), ())),
                        preferred_element_type=jnp.float32)
    h = g * jax.nn.sigmoid(g) * u
    pp = lax.dot_general(h, w3_ref[0], (((1,), (1,)), ((), ())),
                         preferred_element_type=jnp.float32)

    @pl.when(j == 0)
    def _():
        acc_ref[...] = pp

    @pl.when(j > 0)
    def _():
        acc_ref[...] += pp

    @pl.when(j == nh - 1)
    def _():
        ys_ref[...] = acc_ref[...] * wv_ref[...]


def _ffn_grid_spec(P, D, H):
    nt, nh = P // _M, H // _BH
    return pltpu.PrefetchScalarGridSpec(
        num_scalar_prefetch=1,
        grid=(nt, nh),
        in_specs=[
            pl.BlockSpec((_M, D), lambda i, j, eid: (i, 0)),
            pl.BlockSpec((1, _BH, D), lambda i, j, eid: (eid[i], j, 0)),
            pl.BlockSpec((1, _BH, D), lambda i, j, eid: (eid[i], j, 0)),
            pl.BlockSpec((1, D, _BH), lambda i, j, eid: (eid[i], 0, j)),
            pl.BlockSpec((_M, 1), lambda i, j, eid: (i, 0)),
        ],
        out_specs=pl.BlockSpec((_M, D), lambda i, j, eid: (i, 0)),
        scratch_shapes=[pltpu.VMEM((_M, D), jnp.float32)],
    )


def _ffn(eid, xs, W1, W2, W3, wvec):
    P, D = xs.shape
    H = W1.shape[1]
    nh = H // _BH
    return pl.pallas_call(
        functools.partial(_ffn_body, nh=nh),
        grid_spec=_ffn_grid_spec(P, D, H),
        out_shape=jax.ShapeDtypeStruct((P, D), jnp.float32),
        compiler_params=pltpu.CompilerParams(
            dimension_semantics=("arbitrary", "arbitrary")),
    )(eid, xs, W1, W2, W3, wvec[:, None])


def _sc_gather(x2d, src):
    T, D = x2d.shape
    P = src.shape[0]
    info = plsc.get_sparse_core_info()
    nc, ns = info.num_cores, info.num_subcores
    nw = nc * ns
    rw = P // nw
    C = 24
    nch = rw // C
    assert rw % C == 0 and (C % 8) == 0
    mesh = plsc.VectorSubcoreMesh(core_axis_name="c", subcore_axis_name="s")

    @functools.partial(
        pl.kernel, mesh=mesh,
        out_type=jax.ShapeDtypeStruct((P, D), jnp.float32),
        scratch_types=[
            pltpu.VMEM((2, C), jnp.int32),
            pltpu.VMEM((2, C, D), jnp.float32),
            pltpu.SemaphoreType.DMA,
            pltpu.SemaphoreType.DMA,
            pltpu.SemaphoreType.DMA,
            pltpu.SemaphoreType.DMA,
        ])
    def gather_k(x_hbm, src_hbm, out_hbm, idx_v, rows_v, g0, g1, s0, s1):
        wid = lax.axis_index("s") * nc + lax.axis_index("c")
        base = wid * rw
        gsem = (g0, g1)
        ssem = (s0, s1)
        gcp = [None, None]
        scp = [None, None]

        def start_gather(k):
            b = k & 1
            if scp[b] is not None:
                scp[b].wait()
            pltpu.sync_copy(src_hbm.at[pl.ds(base + k * C, C)], idx_v.at[b])
            gcp[b] = pltpu.async_copy(x_hbm.at[idx_v.at[b]], rows_v.at[b],
                                      gsem[b])

        start_gather(0)
        for k in range(nch):
            b = k & 1
            if k + 1 < nch:
                start_gather(k + 1)
            gcp[b].wait()
            scp[b] = pltpu.async_copy(rows_v.at[b],
                                      out_hbm.at[pl.ds(base + k * C, C)],
                                      ssem[b])
        scp[(nch - 1) & 1].wait()
        if nch > 1:
            scp[nch & 1].wait()

    return gather_k(x2d, src)


def _sc_combine(ysw, p0, p1):
    T = p0.shape[0]
    D = ysw.shape[1]
    info = plsc.get_sparse_core_info()
    nc, ns = info.num_cores, info.num_subcores
    nw = nc * ns
    tw = T // nw
    C = 8
    nch = tw // C
    assert tw % C == 0 and (C % 8) == 0
    nvec = D // 16
    mesh = plsc.VectorSubcoreMesh(core_axis_name="c", subcore_axis_name="s")

    @functools.partial(
        pl.kernel, mesh=mesh,
        out_type=jax.ShapeDtypeStruct((T, D), jnp.float32),
        scratch_types=[
            pltpu.VMEM((2, C), jnp.int32),
            pltpu.VMEM((2, C), jnp.int32),
            pltpu.VMEM((2, C, D), jnp.float32),
            pltpu.VMEM((2, C, D), jnp.float32),
            pltpu.SemaphoreType.DMA,
            pltpu.SemaphoreType.DMA,
            pltpu.SemaphoreType.DMA,
            pltpu.SemaphoreType.DMA,
            pltpu.SemaphoreType.DMA,
            pltpu.SemaphoreType.DMA,
        ])
    def combine_k(ysw_hbm, p0_hbm, p1_hbm, out_hbm,
                  i0_v, i1_v, a_v, b_v, ga0, ga1, gb0, gb1, s0, s1):
        wid = lax.axis_index("s") * nc + lax.axis_index("c")
        base = wid * tw
        gasem = (ga0, ga1)
        gbsem = (gb0, gb1)
        ssem = (s0, s1)
        gacp = [None, None]
        gbcp = [None, None]
        scp = [None, None]

        def start_gathers(k):
            b = k & 1
            if scp[b] is not None:
                scp[b].wait()
            pltpu.sync_copy(p0_hbm.at[pl.ds(base + k * C, C)], i0_v.at[b])
            pltpu.sync_copy(p1_hbm.at[pl.ds(base + k * C, C)], i1_v.at[b])
            gacp[b] = pltpu.async_copy(ysw_hbm.at[i0_v.at[b]], a_v.at[b],
                                       gasem[b])
            gbcp[b] = pltpu.async_copy(ysw_hbm.at[i1_v.at[b]], b_v.at[b],
                                       gbsem[b])

        start_gathers(0)
        for k in range(nch):
            b = k & 1
            if k + 1 < nch:
                start_gathers(k + 1)
            gacp[b].wait()
            gbcp[b].wait()
            for r in range(C):
                def add_vec(c, carry, b=b, r=r):
                    sl = pl.ds(c * 16, 16)
                    a_v[b, r, sl] = a_v[b, r, sl] + b_v[b, r, sl]
                    return carry
                lax.fori_loop(0, nvec, add_vec, 0, unroll=8)
            scp[b] = pltpu.async_copy(a_v.at[b],
                                      out_hbm.at[pl.ds(base + k * C, C)],
                                      ssem[b])
        scp[(nch - 1) & 1].wait()
        if nch > 1:
            scp[nch & 1].wait()

    return combine_k(ysw, p0, p1)


_NCH = 2


def kernel(x, Wg, W1, W2, W3):
    B, T, D = x.shape
    H = W1.shape[1]
    P = T * _K + _E * _M
    x2d = x.reshape(B * T, D)

    ts, ti = _gate_topk(x2d, Wg)
    src, wvec, eid, p0, p1 = _routing_metadata(ts, ti, P)

    pc = P // _NCH
    ntc = pc // _M
    ys_parts = []
    for c in range(_NCH):
        sl = slice(c * pc, (c + 1) * pc)
        xs_c = _sc_gather(x2d, src[sl])
        ys_parts.append(
            _ffn(eid[c * ntc:(c + 1) * ntc], xs_c, W1, W2, W3, wvec[sl]))
    ysw = jnp.concatenate(ys_parts, axis=0) if _NCH > 1 else ys_parts[0]
    out = _sc_combine(ysw, p0, p1)
    return out.reshape(B, T, D)


# speedup vs baseline: 1.3001x; 1.3001x over previous
import functools

import jax
import jax.numpy as jnp
from jax import lax
from jax.experimental import pallas as pl
from jax.experimental.pallas import tpu as pltpu
from jax.experimental.pallas import tpu_sc as plsc

_E = 8
_K = 2
_M = 256
_BH = 512
_TB = 256


def _gate_body(x_ref, wg_ref, ts_ref, ti_ref):
    s = lax.dot_general(x_ref[...], wg_ref[...], (((1,), (1,)), ((), ())),
                        preferred_element_type=jnp.float32)
    iota = lax.broadcasted_iota(jnp.int32, s.shape, 1)
    v1 = jnp.max(s, axis=1, keepdims=True)
    i1 = jnp.min(jnp.where(s == v1, iota, _E), axis=1, keepdims=True)
    s2 = jnp.where(iota == i1, -jnp.inf, s)
    v2 = jnp.max(s2, axis=1, keepdims=True)
    i2 = jnp.min(jnp.where(s2 == v2, iota, _E), axis=1, keepdims=True)
    ts_ref[...] = jnp.concatenate([v1, v2], axis=1)
    ti_ref[...] = jnp.concatenate([i1, i2], axis=1)


def _gate_topk(x2d, Wg):
    T, D = x2d.shape
    return pl.pallas_call(
        _gate_body,
        grid=(T // _TB,),
        in_specs=[
            pl.BlockSpec((_TB, D), lambda i: (i, 0)),
            pl.BlockSpec((_E, D), lambda i: (0, 0)),
        ],
        out_specs=[
            pl.BlockSpec((_TB, _K), lambda i: (i, 0)),
            pl.BlockSpec((_TB, _K), lambda i: (i, 0)),
        ],
        out_shape=[
            jax.ShapeDtypeStruct((T, _K), jnp.float32),
            jax.ShapeDtypeStruct((T, _K), jnp.int32),
        ],
    )(x2d, Wg)


def _routing_metadata(ts, ti, P):
    T = ts.shape[0]
    TK = T * _K
    a = ti.reshape(-1)
    oh = (a[:, None] == jnp.arange(_E, dtype=jnp.int32)).astype(jnp.int32)
    csum = jnp.cumsum(oh, axis=0)
    cnt = csum[-1]
    pc = ((cnt + _M - 1) // _M) * _M
    start = jnp.concatenate(
        [jnp.zeros(1, jnp.int32), jnp.cumsum(pc)[:-1].astype(jnp.int32)])
    rank = jnp.take_along_axis(csum, a[:, None], axis=1)[:, 0] - 1
    pos = start[a] + rank
    tok = jnp.arange(TK, dtype=jnp.int32) // _K
    src = (jnp.arange(P, dtype=jnp.int32) % T).at[pos].set(tok)
    wvec = jnp.zeros(P, jnp.float32).at[pos].set(ts.reshape(-1))
    tile_base = jnp.arange(P // _M, dtype=jnp.int32) * _M
    eid = jnp.clip(jnp.searchsorted(start, tile_base, side="right") - 1,
                   0, _E - 1).astype(jnp.int32)
    pos2 = pos.reshape(T, _K)
    return src, wvec, eid, pos2[:, 0], pos2[:, 1]


def _ffn_body(eid_ref, xs_ref, w1_ref, w2_ref, w3_ref, wv_ref, ys_ref,
              acc_ref, *, nh):
    del eid_ref
    j = pl.program_id(1)
    x = xs_ref[...]
    g = lax.dot_general(x, w1_ref[0], (((1,), (1,)), ((), ())),
                        preferred_element_type=jnp.float32)
    u = lax.dot_general(x, w2_ref[0], (((1,), (1,)), ((), ())),
                        preferred_element_type=jnp.float32)
    h = g * jax.nn.sigmoid(g) * u
    pp = lax.dot_general(h, w3_ref[0], (((1,), (1,)), ((), ())),
                         preferred_element_type=jnp.float32)

    @pl.when(j == 0)
    def _():
        acc_ref[...] = pp

    @pl.when(j > 0)
    def _():
        acc_ref[...] += pp

    @pl.when(j == nh - 1)
    def _():
        ys_ref[...] = acc_ref[...] * wv_ref[...]


def _ffn_grid_spec(P, D, H):
    nt, nh = P // _M, H // _BH
    return pltpu.PrefetchScalarGridSpec(
        num_scalar_prefetch=1,
        grid=(nt, nh),
        in_specs=[
            pl.BlockSpec((_M, D), lambda i, j, eid: (i, 0)),
            pl.BlockSpec((1, _BH, D), lambda i, j, eid: (eid[i], j, 0)),
            pl.BlockSpec((1, _BH, D), lambda i, j, eid: (eid[i], j, 0)),
            pl.BlockSpec((1, D, _BH), lambda i, j, eid: (eid[i], 0, j)),
            pl.BlockSpec((_M, 1), lambda i, j, eid: (i, 0)),
        ],
        out_specs=pl.BlockSpec((_M, D), lambda i, j, eid: (i, 0)),
        scratch_shapes=[pltpu.VMEM((_M, D), jnp.float32)],
    )


def _ffn(eid, xs, W1, W2, W3, wvec):
    P, D = xs.shape
    H = W1.shape[1]
    nh = H // _BH
    return pl.pallas_call(
        functools.partial(_ffn_body, nh=nh),
        grid_spec=_ffn_grid_spec(P, D, H),
        out_shape=jax.ShapeDtypeStruct((P, D), jnp.float32),
        compiler_params=pltpu.CompilerParams(
            dimension_semantics=("arbitrary", "arbitrary")),
    )(eid, xs, W1, W2, W3, wvec[:, None])


def _sc_gather(x2d, src):
    T, D = x2d.shape
    P = src.shape[0]
    info = plsc.get_sparse_core_info()
    nc, ns = info.num_cores, info.num_subcores
    nw = nc * ns
    rw = P // nw
    C = 24
    nch = rw // C
    assert rw % C == 0 and (C % 8) == 0
    mesh = plsc.VectorSubcoreMesh(core_axis_name="c", subcore_axis_name="s")

    @functools.partial(
        pl.kernel, mesh=mesh,
        out_type=jax.ShapeDtypeStruct((P, D), jnp.float32),
        scratch_types=[
            pltpu.VMEM((2, C), jnp.int32),
            pltpu.VMEM((2, C, D), jnp.float32),
            pltpu.SemaphoreType.DMA,
            pltpu.SemaphoreType.DMA,
            pltpu.SemaphoreType.DMA,
            pltpu.SemaphoreType.DMA,
        ])
    def gather_k(x_hbm, src_hbm, out_hbm, idx_v, rows_v, g0, g1, s0, s1):
        wid = lax.axis_index("s") * nc + lax.axis_index("c")
        base = wid * rw
        gsem = (g0, g1)
        ssem = (s0, s1)
        gcp = [None, None]
        scp = [None, None]

        def start_gather(k):
            b = k & 1
            if scp[b] is not None:
                scp[b].wait()
            pltpu.sync_copy(src_hbm.at[pl.ds(base + k * C, C)], idx_v.at[b])
            gcp[b] = pltpu.async_copy(x_hbm.at[idx_v.at[b]], rows_v.at[b],
                                      gsem[b])

        start_gather(0)
        for k in range(nch):
            b = k & 1
            if k + 1 < nch:
                start_gather(k + 1)
            gcp[b].wait()
            scp[b] = pltpu.async_copy(rows_v.at[b],
                                      out_hbm.at[pl.ds(base + k * C, C)],
                                      ssem[b])
        scp[(nch - 1) & 1].wait()
        if nch > 1:
            scp[nch & 1].wait()

    return gather_k(x2d, src)


def _sc_combine(ysw, p0, p1):
    T = p0.shape[0]
    D = ysw.shape[1]
    info = plsc.get_sparse_core_info()
    nc, ns = info.num_cores, info.num_subcores
    nw = nc * ns
    tw = T // nw
    C = 8
    nch = tw // C
    assert tw % C == 0 and (C % 8) == 0
    nvec = D // 16
    mesh = plsc.VectorSubcoreMesh(core_axis_name="c", subcore_axis_name="s")

    @functools.partial(
        pl.kernel, mesh=mesh,
        out_type=jax.ShapeDtypeStruct((T, D), jnp.float32),
        scratch_types=[
            pltpu.VMEM((2, C), jnp.int32),
            pltpu.VMEM((2, C), jnp.int32),
            pltpu.VMEM((2, C, D), jnp.float32),
            pltpu.VMEM((2, C, D), jnp.float32),
            pltpu.SemaphoreType.DMA,
            pltpu.SemaphoreType.DMA,
            pltpu.SemaphoreType.DMA,
            pltpu.SemaphoreType.DMA,
            pltpu.SemaphoreType.DMA,
            pltpu.SemaphoreType.DMA,
        ])
    def combine_k(ysw_hbm, p0_hbm, p1_hbm, out_hbm,
                  i0_v, i1_v, a_v, b_v, ga0, ga1, gb0, gb1, s0, s1):
        wid = lax.axis_index("s") * nc + lax.axis_index("c")
        base = wid * tw
        gasem = (ga0, ga1)
        gbsem = (gb0, gb1)
        ssem = (s0, s1)
        gacp = [None, None]
        gbcp = [None, None]
        scp = [None, None]

        def start_gathers(k):
            b = k & 1
            if scp[b] is not None:
                scp[b].wait()
            pltpu.sync_copy(p0_hbm.at[pl.ds(base + k * C, C)], i0_v.at[b])
            pltpu.sync_copy(p1_hbm.at[pl.ds(base + k * C, C)], i1_v.at[b])
            gacp[b] = pltpu.async_copy(ysw_hbm.at[i0_v.at[b]], a_v.at[b],
                                       gasem[b])
            gbcp[b] = pltpu.async_copy(ysw_hbm.at[i1_v.at[b]], b_v.at[b],
                                       gbsem[b])

        start_gathers(0)
        for k in range(nch):
            b = k & 1
            if k + 1 < nch:
                start_gathers(k + 1)
            gacp[b].wait()
            gbcp[b].wait()
            for r in range(C):
                def add_vec(c, carry, b=b, r=r):
                    sl = pl.ds(c * 16, 16)
                    a_v[b, r, sl] = a_v[b, r, sl] + b_v[b, r, sl]
                    return carry
                lax.fori_loop(0, nvec, add_vec, 0, unroll=8)
            scp[b] = pltpu.async_copy(a_v.at[b],
                                      out_hbm.at[pl.ds(base + k * C, C)],
                                      ssem[b])
        scp[(nch - 1) & 1].wait()
        if nch > 1:
            scp[nch & 1].wait()

    return combine_k(ysw, p0, p1)


_NCH = 1


def kernel(x, Wg, W1, W2, W3):
    B, T, D = x.shape
    H = W1.shape[1]
    P = T * _K + _E * _M
    x2d = x.reshape(B * T, D)

    ts, ti = _gate_topk(x2d, Wg)
    src, wvec, eid, p0, p1 = _routing_metadata(ts, ti, P)

    pc = P // _NCH
    ntc = pc // _M
    ys_parts = []
    for c in range(_NCH):
        sl = slice(c * pc, (c + 1) * pc)
        xs_c = _sc_gather(x2d, src[sl])
        ys_parts.append(
            _ffn(eid[c * ntc:(c + 1) * ntc], xs_c, W1, W2, W3, wvec[sl]))
    ysw = jnp.concatenate(ys_parts, axis=0) if _NCH > 1 else ys_parts[0]
    out = _sc_combine(ysw, p0, p1)
    return out.reshape(B, T, D)
